# baseline (speedup 1.0000x reference)
import functools

import jax
import jax.numpy as jnp
from jax import lax
from jax.experimental import pallas as pl
from jax.experimental.pallas import tpu as pltpu
from jax.experimental.pallas import tpu_sc as plsc

NUM_VIRTUAL_TOKENS = 128
ROW_DIM = 131072
CHUNK = 2048
NCH = ROW_DIM // CHUNK
B_TOTAL = 512
LANES = 16

_info = plsc.get_sparse_core_info()
NC, NS = _info.num_cores, _info.num_subcores
NW = NC * NS
B_PER_W = B_TOTAL // NW


@functools.partial(
    pl.kernel,
    out_type=jax.ShapeDtypeStruct((B_TOTAL, NCH, CHUNK), jnp.float32),
    mesh=plsc.VectorSubcoreMesh(core_axis_name="c", subcore_axis_name="s"),
    scratch_types=[
        pltpu.VMEM((B_PER_W,), jnp.int32),
        pltpu.VMEM((LANES, CHUNK), jnp.float32),
        pltpu.SemaphoreType.DMA,
    ],
)
def _gather_kernel(prefix_hbm, table_hbm, out_hbm, idx_v, buf_v, sem):
    wid = lax.axis_index("s") * NC + lax.axis_index("c")
    base_b = wid * B_PER_W
    pltpu.sync_copy(prefix_hbm.at[pl.ds(base_b, B_PER_W)], idx_v)
    ids = idx_v[...]
    for j in range(NCH):
        src = ids * NCH + j
        pltpu.async_copy(table_hbm.at[src], buf_v, sem).wait()
        pltpu.sync_copy(buf_v, out_hbm.at[pl.ds(base_b, B_PER_W), j])


def kernel(prefix, table):
    pfx = prefix.reshape(B_TOTAL)
    tbl = table.reshape(NUM_VIRTUAL_TOKENS * NCH, CHUNK)
    out = _gather_kernel(pfx, tbl)
    return out.reshape(prefix.shape[0], prefix.shape[1], ROW_DIM)


# speedup vs baseline: 1.1089x; 1.1089x over previous
import functools

import jax
import jax.numpy as jnp
from jax import lax
from jax.experimental import pallas as pl
from jax.experimental.pallas import tpu as pltpu
from jax.experimental.pallas import tpu_sc as plsc

NUM_VIRTUAL_TOKENS = 128
ROW_DIM = 131072
CHUNK = 2048
NCH = ROW_DIM // CHUNK
B_TOTAL = 512
LANES = 16

_info = plsc.get_sparse_core_info()
NC, NS = _info.num_cores, _info.num_subcores
NW = NC * NS
B_PER_W = B_TOTAL // NW


@functools.partial(
    pl.kernel,
    out_type=jax.ShapeDtypeStruct((B_TOTAL, NCH, CHUNK), jnp.float32),
    mesh=plsc.VectorSubcoreMesh(core_axis_name="c", subcore_axis_name="s"),
    scratch_types=[
        pltpu.VMEM((B_PER_W,), jnp.int32),
        pltpu.VMEM((LANES, CHUNK), jnp.float32),
        pltpu.VMEM((LANES, CHUNK), jnp.float32),
        pltpu.SemaphoreType.DMA,
        pltpu.SemaphoreType.DMA,
        pltpu.SemaphoreType.DMA,
        pltpu.SemaphoreType.DMA,
    ],
)
def _gather_kernel(prefix_hbm, table_hbm, out_hbm, idx_v,
                   buf0, buf1, gs0, gs1, ss0, ss1):
    wid = lax.axis_index("s") * NC + lax.axis_index("c")
    base_b = wid * B_PER_W
    pltpu.sync_copy(prefix_hbm.at[pl.ds(base_b, B_PER_W)], idx_v)
    srcbase = idx_v[...] * NCH
    buf, gsem, ssem = (buf0, buf1), (gs0, gs1), (ss0, ss1)

    def start_gather(j):
        return pltpu.async_copy(
            table_hbm.at[srcbase + j], buf[j % 2], gsem[j % 2])

    gather = [start_gather(0), None]
    scatter = [None, None]
    for j in range(NCH):
        nxt = j + 1
        if nxt < NCH:
            if scatter[nxt % 2] is not None:
                scatter[nxt % 2].wait()
            gather[nxt % 2] = start_gather(nxt)
        gather[j % 2].wait()
        scatter[j % 2] = pltpu.async_copy(
            buf[j % 2], out_hbm.at[pl.ds(base_b, B_PER_W), j], ssem[j % 2])
    scatter[0].wait()
    scatter[1].wait()


def kernel(prefix, table):
    pfx = prefix.reshape(B_TOTAL)
    tbl = table.reshape(NUM_VIRTUAL_TOKENS * NCH, CHUNK)
    out = _gather_kernel(pfx, tbl)
    return out.reshape(prefix.shape[0], prefix.shape[1], ROW_DIM)


# speedup vs baseline: 2.3233x; 2.0952x over previous
import functools

import jax
import jax.numpy as jnp
from jax import lax
from jax.experimental import pallas as pl
from jax.experimental.pallas import tpu as pltpu
from jax.experimental.pallas import tpu_sc as plsc

BATCH = 4
PREFIX_LEN = 128
ROW_DIM = 131072
CHUNK = 2048
NCH = ROW_DIM // CHUNK
B_TOTAL = BATCH * PREFIX_LEN
LANES = 16

_info = plsc.get_sparse_core_info()
NC, NS = _info.num_cores, _info.num_subcores
NW = NC * NS
B_PER_W = B_TOTAL // NW


@functools.partial(
    pl.kernel,
    out_type=jax.ShapeDtypeStruct((BATCH, PREFIX_LEN, ROW_DIM), jnp.float32),
    mesh=plsc.VectorSubcoreMesh(core_axis_name="c", subcore_axis_name="s"),
    scratch_types=[
        pltpu.VMEM((B_PER_W,), jnp.int32),
        pltpu.VMEM((LANES, CHUNK), jnp.float32),
        pltpu.VMEM((LANES, CHUNK), jnp.float32),
        pltpu.SemaphoreType.DMA,
        pltpu.SemaphoreType.DMA,
        pltpu.SemaphoreType.DMA,
        pltpu.SemaphoreType.DMA,
    ],
)
def _gather_kernel(prefix_hbm, table_hbm, out_hbm, idx_v,
                   buf0, buf1, gs0, gs1, ss0, ss1):
    wid = lax.axis_index("s") * NC + lax.axis_index("c")
    base_b = wid * B_PER_W
    bi = base_b // PREFIX_LEN
    pos0 = base_b % PREFIX_LEN
    pltpu.sync_copy(prefix_hbm.at[pl.ds(base_b, B_PER_W)], idx_v)
    ids = idx_v[...]
    buf, gsem, ssem = (buf0, buf1), (gs0, gs1), (ss0, ss1)

    def start_gather(j):
        return pltpu.async_copy(
            table_hbm.at[ids, pl.ds(j * CHUNK, CHUNK)], buf[j % 2],
            gsem[j % 2])

    gather = [start_gather(0), None]
    scatter = [None, None]
    for j in range(NCH):
        nxt = j + 1
        if nxt < NCH:
            if scatter[nxt % 2] is not None:
                scatter[nxt % 2].wait()
            gather[nxt % 2] = start_gather(nxt)
        gather[j % 2].wait()
        scatter[j % 2] = pltpu.async_copy(
            buf[j % 2],
            out_hbm.at[bi, pl.ds(pos0, B_PER_W), pl.ds(j * CHUNK, CHUNK)],
            ssem[j % 2])
    scatter[0].wait()
    scatter[1].wait()


def kernel(prefix, table):
    return _gather_kernel(prefix.reshape(B_TOTAL), table)


# speedup vs baseline: 2.3803x; 1.0245x over previous
import functools

import jax
import jax.numpy as jnp
from jax import lax
from jax.experimental import pallas as pl
from jax.experimental.pallas import tpu as pltpu
from jax.experimental.pallas import tpu_sc as plsc

BATCH = 4
PREFIX_LEN = 128
ROW_DIM = 131072
CHUNK = 2048
NCH = ROW_DIM // CHUNK
B_TOTAL = BATCH * PREFIX_LEN
LANES = 16

_info = plsc.get_sparse_core_info()
NC, NS = _info.num_cores, _info.num_subcores
NW = NC * NS
B_PER_W = B_TOTAL // NW


@functools.partial(
    pl.kernel,
    out_type=jax.ShapeDtypeStruct((BATCH, PREFIX_LEN, ROW_DIM), jnp.float32),
    mesh=plsc.VectorSubcoreMesh(core_axis_name="c", subcore_axis_name="s"),
    scratch_types=[
        pltpu.VMEM((B_PER_W,), jnp.int32),
        pltpu.VMEM((LANES, CHUNK), jnp.float32),
        pltpu.VMEM((LANES, CHUNK), jnp.float32),
        pltpu.VMEM((LANES, CHUNK), jnp.float32),
        pltpu.SemaphoreType.DMA,
        pltpu.SemaphoreType.DMA,
        pltpu.SemaphoreType.DMA,
        pltpu.SemaphoreType.DMA,
        pltpu.SemaphoreType.DMA,
        pltpu.SemaphoreType.DMA,
    ],
)
def _gather_kernel(prefix_hbm, table_hbm, out_hbm, idx_v,
                   buf0, buf1, buf2, gs0, gs1, gs2, ss0, ss1, ss2):
    wid = lax.axis_index("s") * NC + lax.axis_index("c")
    base_b = wid * B_PER_W
    bi = base_b // PREFIX_LEN
    pos0 = base_b % PREFIX_LEN
    pltpu.sync_copy(prefix_hbm.at[pl.ds(base_b, B_PER_W)], idx_v)
    ids = idx_v[...]
    buf, gsem, ssem = (buf0, buf1, buf2), (gs0, gs1, gs2), (ss0, ss1, ss2)
    D = len(buf)

    def start_gather(j):
        return pltpu.async_copy(
            table_hbm.at[ids, pl.ds(j * CHUNK, CHUNK)], buf[j % D],
            gsem[j % D])

    gather = [None] * D
    scatter = [None] * D
    for i in range(D - 1):
        gather[i] = start_gather(i)
    for j in range(NCH):
        nxt = j + D - 1
        if nxt < NCH:
            if scatter[nxt % D] is not None:
                scatter[nxt % D].wait()
            gather[nxt % D] = start_gather(nxt)
        gather[j % D].wait()
        scatter[j % D] = pltpu.async_copy(
            buf[j % D],
            out_hbm.at[bi, pl.ds(pos0, B_PER_W), pl.ds(j * CHUNK, CHUNK)],
            ssem[j % D])
    for i in range(D):
        if scatter[i] is not None:
            scatter[i].wait()


def kernel(prefix, table):
    return _gather_kernel(prefix.reshape(B_TOTAL), table)


# speedup vs baseline: 3.1052x; 1.3046x over previous
import functools

import jax
import jax.numpy as jnp
from jax import lax
from jax.experimental import pallas as pl
from jax.experimental.pallas import tpu as pltpu
from jax.experimental.pallas import tpu_sc as plsc

BATCH = 4
PREFIX_LEN = 128
NUM_VIRTUAL_TOKENS = 128
ROW_DIM = 131072
CHUNK = 2048
NCH = ROW_DIM // CHUNK
B_TOTAL = BATCH * PREFIX_LEN
LANES = 16
SD = 3

_info = plsc.get_sparse_core_info()
NC, NS = _info.num_cores, _info.num_subcores
NW = NC * NS
B_PER_W = B_TOTAL // NW
ROWS_PER_TILE = NUM_VIRTUAL_TOKENS // NS


@functools.partial(
    pl.kernel,
    out_type=jax.ShapeDtypeStruct((BATCH, PREFIX_LEN, ROW_DIM), jnp.float32),
    mesh=plsc.VectorSubcoreMesh(core_axis_name="c", subcore_axis_name="s"),
    scratch_types=[
        pltpu.VMEM((B_PER_W,), jnp.int32),
        pltpu.VMEM_SHARED((NUM_VIRTUAL_TOKENS, CHUNK), jnp.float32),
        pltpu.VMEM_SHARED((NUM_VIRTUAL_TOKENS, CHUNK), jnp.float32),
        pltpu.VMEM_SHARED((NUM_VIRTUAL_TOKENS, CHUNK), jnp.float32),
        pltpu.SemaphoreType.DMA,
        pltpu.SemaphoreType.DMA,
        pltpu.SemaphoreType.DMA,
        pltpu.SemaphoreType.DMA,
        pltpu.SemaphoreType.DMA,
        pltpu.SemaphoreType.DMA,
    ],
)
def _gather_kernel(prefix_hbm, table_hbm, out_hbm, idx_v,
                   sh0, sh1, sh2, ls0, ls1, ls2, ws0, ws1, ws2):
    sid = lax.axis_index("s")
    wid = sid * NC + lax.axis_index("c")
    base_b = wid * B_PER_W
    bi = base_b // PREFIX_LEN
    pos0 = base_b % PREFIX_LEN
    pltpu.sync_copy(prefix_hbm.at[pl.ds(base_b, B_PER_W)], idx_v)
    ids = idx_v[...]
    row0 = sid * ROWS_PER_TILE
    sh, lsem, wsem = (sh0, sh1, sh2), (ls0, ls1, ls2), (ws0, ws1, ws2)

    def start_load(j, k):
        pltpu.async_copy(
            table_hbm.at[pl.ds(row0, ROWS_PER_TILE),
                         pl.ds(j * CHUNK, CHUNK)],
            sh[k].at[pl.ds(row0, ROWS_PER_TILE)],
            lsem[k])

    def drain_load(k):
        pltpu.make_async_copy(
            table_hbm.at[pl.ds(0, ROWS_PER_TILE), pl.ds(0, CHUNK)],
            sh[k].at[pl.ds(row0, ROWS_PER_TILE)],
            lsem[k]).wait()

    def fire_writes(j, k):
        for i in range(B_PER_W):
            v = ids[i]
            pltpu.async_copy(
                sh[k].at[v],
                out_hbm.at[bi, pos0 + i, pl.ds(j * CHUNK, CHUNK)],
                wsem[k])

    def drain_writes(k):
        pltpu.make_async_copy(
            table_hbm.at[pl.ds(0, B_PER_W), pl.ds(0, CHUNK)],
            out_hbm.at[bi, pl.ds(pos0, B_PER_W), pl.ds(0, CHUNK)],
            wsem[k]).wait()

    start_load(0, 0)
    start_load(1, 1)
    start_load(2, 2)
    drain_load(0)
    plsc.subcore_barrier()
    fire_writes(0, 0)

    def body(g, carry):
        for k0 in range(SD):
            j = SD * g + k0 + 1
            ka = (k0 + 1) % SD
            drain_load(ka)
            plsc.subcore_barrier()
            fire_writes(j, ka)
            drain_writes(k0)
            plsc.subcore_barrier()
            @pl.when(j + 2 < NCH)
            def _():
                start_load(j + 2, k0)
        return carry

    lax.fori_loop(0, (NCH - 1) // SD, body, 0)
    drain_writes((NCH - 1) % SD)


def kernel(prefix, table):
    return _gather_kernel(prefix.reshape(B_TOTAL), table)


# speedup vs baseline: 3.1926x; 1.0281x over previous
import functools

import jax
import jax.numpy as jnp
from jax import lax
from jax.experimental import pallas as pl
from jax.experimental.pallas import tpu as pltpu
from jax.experimental.pallas import tpu_sc as plsc

BATCH = 4
PREFIX_LEN = 128
NUM_VIRTUAL_TOKENS = 128
ROW_DIM = 131072
CHUNK = 2048
NCH = ROW_DIM // CHUNK
B_TOTAL = BATCH * PREFIX_LEN
LANES = 16
SD = 3

_info = plsc.get_sparse_core_info()
NC, NS = _info.num_cores, _info.num_subcores
CH_PER_CORE = NCH // NC
B_PER_TILE = B_TOTAL // NS
ROWS_PER_TILE = NUM_VIRTUAL_TOKENS // NS


@functools.partial(
    pl.kernel,
    out_type=jax.ShapeDtypeStruct((BATCH, PREFIX_LEN, ROW_DIM), jnp.float32),
    mesh=plsc.VectorSubcoreMesh(core_axis_name="c", subcore_axis_name="s"),
    scratch_types=[
        pltpu.VMEM((B_PER_TILE,), jnp.int32),
        pltpu.VMEM_SHARED((NUM_VIRTUAL_TOKENS, CHUNK), jnp.float32),
        pltpu.VMEM_SHARED((NUM_VIRTUAL_TOKENS, CHUNK), jnp.float32),
        pltpu.VMEM_SHARED((NUM_VIRTUAL_TOKENS, CHUNK), jnp.float32),
        pltpu.SemaphoreType.DMA,
        pltpu.SemaphoreType.DMA,
        pltpu.SemaphoreType.DMA,
        pltpu.SemaphoreType.DMA,
        pltpu.SemaphoreType.DMA,
        pltpu.SemaphoreType.DMA,
    ],
)
def _gather_kernel(prefix_hbm, table_hbm, out_hbm, idx_v,
                   sh0, sh1, sh2, ls0, ls1, ls2, ws0, ws1, ws2):
    sid = lax.axis_index("s")
    cid = lax.axis_index("c")
    j0 = cid * CH_PER_CORE
    base_b = sid * B_PER_TILE
    bi = base_b // PREFIX_LEN
    pos0 = base_b % PREFIX_LEN
    pltpu.sync_copy(prefix_hbm.at[pl.ds(base_b, B_PER_TILE)], idx_v)
    ids = (idx_v[pl.ds(0, LANES)], idx_v[pl.ds(LANES, LANES)])
    row0 = sid * ROWS_PER_TILE
    sh, lsem, wsem = (sh0, sh1, sh2), (ls0, ls1, ls2), (ws0, ws1, ws2)

    def start_load(r, k):
        pltpu.async_copy(
            table_hbm.at[pl.ds(row0, ROWS_PER_TILE),
                         pl.ds((j0 + r) * CHUNK, CHUNK)],
            sh[k].at[pl.ds(row0, ROWS_PER_TILE)],
            lsem[k])

    def drain_load(k):
        pltpu.make_async_copy(
            table_hbm.at[pl.ds(0, ROWS_PER_TILE), pl.ds(0, CHUNK)],
            sh[k].at[pl.ds(row0, ROWS_PER_TILE)],
            lsem[k]).wait()

    def fire_writes(r, k):
        for i in range(B_PER_TILE):
            v = ids[i // LANES][i % LANES]
            pltpu.async_copy(
                sh[k].at[v],
                out_hbm.at[bi, pos0 + i, pl.ds((j0 + r) * CHUNK, CHUNK)],
                wsem[k])

    def drain_writes(k):
        pltpu.make_async_copy(
            table_hbm.at[pl.ds(0, B_PER_TILE), pl.ds(0, CHUNK)],
            out_hbm.at[bi, pl.ds(pos0, B_PER_TILE), pl.ds(0, CHUNK)],
            wsem[k]).wait()

    start_load(0, 0)
    start_load(1, 1)
    start_load(2, 2)
    drain_load(0)
    plsc.subcore_barrier()
    fire_writes(0, 0)

    def body(g, carry):
        for k0 in range(SD):
            r = SD * g + k0 + 1
            ka = (k0 + 1) % SD
            drain_load(ka)
            plsc.subcore_barrier()
            fire_writes(r, ka)
            drain_writes(k0)
            plsc.subcore_barrier()
            @pl.when(r + 2 < CH_PER_CORE)
            def _():
                start_load(r + 2, k0)
        return carry

    lax.fori_loop(0, (CH_PER_CORE - 2) // SD, body, 0)
    drain_load(1)
    plsc.subcore_barrier()
    fire_writes(CH_PER_CORE - 1, 1)
    drain_writes(0)
    drain_writes(1)


def kernel(prefix, table):
    return _gather_kernel(prefix.reshape(B_TOTAL), table)


# speedup vs baseline: 3.5320x; 1.1063x over previous
import functools

import jax
import jax.numpy as jnp
from jax import lax
from jax.experimental import pallas as pl
from jax.experimental.pallas import tpu as pltpu
from jax.experimental.pallas import tpu_sc as plsc

BATCH = 4
PREFIX_LEN = 128
NUM_VIRTUAL_TOKENS = 128
ROW_DIM = 131072
CHUNK = 2048
NCH = ROW_DIM // CHUNK
B_TOTAL = BATCH * PREFIX_LEN
LANES = 16
SD = 3

_info = plsc.get_sparse_core_info()
NC, NS = _info.num_cores, _info.num_subcores
CH_PER_CORE = NCH // NC
B_PER_TILE = B_TOTAL // NS
ROWS_PER_TILE = NUM_VIRTUAL_TOKENS // NS


@functools.partial(
    pl.kernel,
    out_type=jax.ShapeDtypeStruct((BATCH, PREFIX_LEN, ROW_DIM), jnp.float32),
    mesh=plsc.VectorSubcoreMesh(core_axis_name="c", subcore_axis_name="s"),
    scratch_types=[
        pltpu.VMEM((B_PER_TILE,), jnp.int32),
        pltpu.VMEM((LANES, CHUNK), jnp.float32),
        pltpu.VMEM_SHARED((NUM_VIRTUAL_TOKENS, CHUNK), jnp.float32),
        pltpu.VMEM_SHARED((NUM_VIRTUAL_TOKENS, CHUNK), jnp.float32),
        pltpu.VMEM_SHARED((NUM_VIRTUAL_TOKENS, CHUNK), jnp.float32),
        pltpu.SemaphoreType.DMA,
        pltpu.SemaphoreType.DMA,
        pltpu.SemaphoreType.DMA,
        pltpu.SemaphoreType.DMA,
        pltpu.SemaphoreType.DMA,
        pltpu.SemaphoreType.DMA,
        pltpu.SemaphoreType.DMA,
        pltpu.SemaphoreType.DMA,
    ],
)
def _gather_kernel(prefix_hbm, table_hbm, out_hbm, idx_v, bb,
                   sh0, sh1, sh2, ls0, ls1, ls2, ws0, ws1, ws2,
                   bcsem, bssem):
    sid = lax.axis_index("s")
    cid = lax.axis_index("c")
    j0 = cid * CH_PER_CORE
    base_b = sid * B_PER_TILE
    bi = base_b // PREFIX_LEN
    pos0 = base_b % PREFIX_LEN
    pltpu.sync_copy(prefix_hbm.at[pl.ds(base_b, B_PER_TILE)], idx_v)
    ids = (idx_v[pl.ds(0, LANES)], idx_v[pl.ds(LANES, LANES)])
    row0 = sid * ROWS_PER_TILE
    sh, lsem, wsem = (sh0, sh1, sh2), (ls0, ls1, ls2), (ws0, ws1, ws2)

    def start_load(r, k):
        pltpu.async_copy(
            table_hbm.at[pl.ds(row0, ROWS_PER_TILE),
                         pl.ds((j0 + r) * CHUNK, CHUNK)],
            sh[k].at[pl.ds(row0, ROWS_PER_TILE)],
            lsem[k])

    def drain_load(k):
        pltpu.make_async_copy(
            table_hbm.at[pl.ds(0, ROWS_PER_TILE), pl.ds(0, CHUNK)],
            sh[k].at[pl.ds(row0, ROWS_PER_TILE)],
            lsem[k]).wait()

    def fire_writes(r, k, first=False):
        for i in range(LANES):
            v = ids[0][i]
            pltpu.async_copy(
                sh[k].at[v],
                out_hbm.at[bi, pos0 + i, pl.ds((j0 + r) * CHUNK, CHUNK)],
                wsem[k])
        if not first:
            pltpu.make_async_copy(
                table_hbm.at[pl.ds(0, LANES), pl.ds(0, CHUNK)],
                bb, bssem).wait()
        for i in range(LANES):
            v = ids[1][i]
            pltpu.async_copy(sh[k].at[v], bb.at[i], bcsem)
        pltpu.make_async_copy(
            table_hbm.at[pl.ds(0, LANES), pl.ds(0, CHUNK)],
            bb, bcsem).wait()
        pltpu.async_copy(
            bb,
            out_hbm.at[bi, pl.ds(pos0 + LANES, LANES),
                       pl.ds((j0 + r) * CHUNK, CHUNK)],
            bssem)

    def drain_writes(k):
        pltpu.make_async_copy(
            table_hbm.at[pl.ds(0, LANES), pl.ds(0, CHUNK)],
            out_hbm.at[bi, pl.ds(pos0, LANES), pl.ds(0, CHUNK)],
            wsem[k]).wait()

    start_load(0, 0)
    start_load(1, 1)
    start_load(2, 2)
    drain_load(0)
    plsc.subcore_barrier()
    fire_writes(0, 0, first=True)

    def body(g, carry):
        for k0 in range(SD):
            r = SD * g + k0 + 1
            ka = (k0 + 1) % SD
            drain_load(ka)
            plsc.subcore_barrier()
            fire_writes(r, ka)
            drain_writes(k0)
            plsc.subcore_barrier()
            @pl.when(r + 2 < CH_PER_CORE)
            def _():
                start_load(r + 2, k0)
        return carry

    lax.fori_loop(0, (CH_PER_CORE - 2) // SD, body, 0)
    drain_load(1)
    plsc.subcore_barrier()
    fire_writes(CH_PER_CORE - 1, 1)
    drain_writes(0)
    drain_writes(1)
    pltpu.make_async_copy(
        table_hbm.at[pl.ds(0, LANES), pl.ds(0, CHUNK)],
        bb, bssem).wait()


def kernel(prefix, table):
    return _gather_kernel(prefix.reshape(B_TOTAL), table)
